# baseline (speedup 1.0000x reference)
import jax
import jax.numpy as jnp
from jax.experimental import pallas as pl

_F32 = jnp.float32
_DEF = jax.lax.Precision.DEFAULT


def _leaky(v):
    return jnp.where(v >= 0, v, 0.01 * v)


def _mm(a, b):
    return jax.lax.dot_general(a, b, (((1,), (0,)), ((), ())),
                               precision=_DEF, preferred_element_type=_F32)


def _mmR(a, w):
    return jax.lax.dot_general(a, w, (((1,), (1,)), ((), ())),
                               precision=_DEF, preferred_element_type=_F32)


def _mmT(a, b):
    return jax.lax.dot_general(a, b, (((0,), (0,)), ((), ())),
                               precision=_DEF, preferred_element_type=_F32)


def _mlpR(x, w1, b1, w2, b2):
    h = _leaky(_mmR(x, w1) + b1)
    return _leaky(_mmR(h, w2) + b2)


_X0 = 0
_OI = 32
_NB = 36
_CFW1 = 68
_CFW2 = 196
_FW1 = 324
_FW2 = 452
_GW1 = 580
_GW2 = 708
_IW1 = 836
_IW2 = 964
_OSW = 1092
_CB1 = 1093
_CB2 = 1094
_FB1 = 1096
_FB2 = 1097
_GB1 = 1098
_GB2 = 1099
_IB1 = 1100
_IB2 = 1101
_OSB = 1102
_NROWS = 1104


def _body(pa_ref, out_ref):
    X0 = pa_ref[_X0:_X0 + 32, :]
    oif = pa_ref[_OI:_OI + 4, 0:8]
    nbrf = pa_ref[_NB:_NB + 32, 0:4]

    fw1 = pa_ref[_FW1:_FW1 + 128, :]
    fw2 = pa_ref[_FW2:_FW2 + 128, :]
    gw1 = pa_ref[_GW1:_GW1 + 128, :]
    gw2 = pa_ref[_GW2:_GW2 + 128, :]
    iw1 = pa_ref[_IW1:_IW1 + 128, :]
    iw2 = pa_ref[_IW2:_IW2 + 128, :]
    fb1 = pa_ref[_FB1:_FB1 + 1, :]
    fb2 = pa_ref[_FB2:_FB2 + 1, :]
    gb1 = pa_ref[_GB1:_GB1 + 1, :]
    gb2 = pa_ref[_GB2:_GB2 + 1, :]
    ib1 = pa_ref[_IB1:_IB1 + 1, :]
    ib2 = pa_ref[_IB2:_IB2 + 1, :]

    h = _leaky(_mmR(X0, pa_ref[_CFW1:_CFW1 + 128, :])
               + pa_ref[_CB1:_CB1 + 1, :])
    cfw2 = pa_ref[_CFW2:_CFW2 + 128, :]
    cb2 = pa_ref[_CB2:_CB2 + 2, :]
    F0 = _leaky(_mmR(h[:, 0:64], cfw2[:, 0:64]) + cb2[0:1, :])
    F1 = _leaky(_mmR(h[:, 64:128], cfw2[:, 64:128]) + cb2[1:2, :])
    F = jnp.concatenate([F0, F1], axis=1)

    _i32 = jnp.int32
    k_col = jax.lax.broadcasted_iota(_i32, (4, 1), 0).astype(_F32) * 8.0
    g_oi = oif + k_col
    iota483 = jax.lax.broadcasted_iota(_i32, (4, 8, 32), 2).astype(_F32)
    O32 = (iota483 == g_oi[:, :, None]).astype(_F32).reshape(32, 32)
    iota432 = jax.lax.broadcasted_iota(_i32, (4, 32), 1).astype(_F32)

    nb_all = _mm(O32, nbrf)
    Gs, moAs, moBs, Hps = [], [], [], []
    mAeq, mBeq, mTeq = [None], [None], [None]
    nb_pads = []
    for p in range(8):
        ip = oif[:, p:p + 1]
        gp = g_oi[:, p:p + 1]
        Hp = (iota432 == gp).astype(_F32)
        Sp = (iota432 == k_col + float(p)).astype(_F32)
        nb_sel = _mm(Sp, nb_all)
        nb_pad = jnp.concatenate(
            [nb_sel, jnp.full((4, 4), -100.0, _F32)], axis=1)
        g_nb = nb_pad + k_col
        Nb = (iota483 == g_nb[:, :, None]).astype(_F32).reshape(32, 32)
        Gs.append(jnp.concatenate([O32, Nb, Hp], axis=0))
        Hps.append(Hp)
        nb_pads.append(nb_pad)
        moAs.append((oif != ip).astype(_F32))
        moBs.append(((nb_pad != ip) & (nb_pad >= 0)).astype(_F32))
    for p in range(1, 8):
        ipm1 = oif[:, p - 1:p]
        mAeq.append((oif == ipm1).astype(_F32))
        mBeq.append((nb_pads[p] == ipm1).astype(_F32))
        mTeq.append((oif[:, p:p + 1] == ipm1).astype(_F32))

    Fstate = F
    Rpre = _mm(Gs[0], F)
    d_prev = None
    for p in range(8):
        if p == 0:
            A = Rpre[0:32]
            B = Rpre[32:64]
            T = Rpre[64:68]
        else:
            dbc = d_prev[:, None, :]
            A = Rpre[0:32] + (mAeq[p][:, :, None] * dbc).reshape(32, 256)
            B = Rpre[32:64] + (mBeq[p][:, :, None] * dbc).reshape(32, 256)
            T = Rpre[64:68] + mTeq[p] * d_prev
        Te = T[:, None, :]
        Xa = (Te * A.reshape(4, 8, 256)).reshape(32, 256)
        Xb = (Te * B.reshape(4, 8, 256)).reshape(32, 256)
        X = jnp.concatenate([Xa, Xb], axis=0)
        Xj = jnp.concatenate([X[:, :128], X[:, 128:]], axis=0)
        U = _mlpR(Xj, fw1, fb1, fw2, fb2)

        moA = moAs[p]
        moB = moBs[p]
        d0 = (jnp.sum(U[0:32].reshape(4, 8, 128) * moA[:, :, None], axis=1)
              + jnp.sum(U[32:64].reshape(4, 8, 128) * moB[:, :, None], axis=1))
        d1 = (jnp.sum(U[64:96].reshape(4, 8, 128) * moA[:, :, None], axis=1)
              + jnp.sum(U[96:128].reshape(4, 8, 128) * moB[:, :, None], axis=1))
        d = jnp.concatenate([d0, d1], axis=1)
        if p >= 1:
            Fstate = Fstate + _mmT(Hps[p - 1], d_prev)
        if p < 7:
            Rpre = _mm(Gs[p + 1], Fstate)
        d_prev = d
    F = Fstate + _mmT(Hps[7], d_prev)

    X3 = jnp.concatenate([F[:, :128], F[:, 128:]], axis=0)
    U3 = _mlpR(X3, gw1, gb1, gw2, gb2)
    Usum = U3[0:32] + U3[32:64]
    cnt = _mmT(O32, jnp.ones((32, 1), _F32))
    Ff = X0 + cnt * Usum

    Ffstate = Ff
    Wgpre = _mm(O32, Ff)
    s_prev = None
    for p in range(8):
        if p == 0:
            Wg = Wgpre
        else:
            Wg = Wgpre + (mAeq[p][:, :, None]
                          * s_prev[:, None, :]).reshape(32, 128)
        V = _mlpR(Wg, iw1, ib1, iw2, ib2)
        s = jnp.sum(V.reshape(4, 8, 128) * moAs[p][:, :, None], axis=1)
        if p >= 1:
            Ffstate = Ffstate + _mmT(Hps[p - 1], s_prev)
        if p < 7:
            Wgpre = _mm(O32, Ffstate)
        s_prev = s
    Ff = Ffstate + _mmT(Hps[7], s_prev)

    C = jnp.sum(O32.reshape(4, 8, 32), axis=1)
    outf = _mm(C, Ff)
    logits = (jnp.sum(outf * pa_ref[_OSW:_OSW + 1, :], axis=1, keepdims=True)
              + pa_ref[_OSB, 0])
    sc = jax.nn.sigmoid(logits)

    a = F[:, :128]
    b = F[:, 128:]
    aa = jnp.sum(a * a, axis=1, keepdims=True)
    ab = jnp.sum(a * b, axis=1, keepdims=True)
    bb = jnp.sum(b * b, axis=1, keepdims=True)
    terms = (aa - 1.0) ** 2 + 2.0 * ab * ab + (bb - 1.0) ** 2
    loss = jnp.sum(terms, axis=0, keepdims=True) / 4.0

    out_ref[:] = jnp.concatenate(
        [jnp.broadcast_to(sc, (4, 128)), jnp.broadcast_to(loss, (4, 128))],
        axis=0)


def kernel(outfit_items_posi, items_feature_posi, items_neighbor_posi,
           items_factors,
           cf_W1, cf_b1, cf_W2, cf_b2,
           f2f_W1, f2f_b1, f2f_W2, f2f_b2,
           f2i_W1, f2i_b1, f2i_W2, f2i_b2,
           i2i_W1, i2i_b1, i2i_W2, i2i_b2,
           o2s_W, o2s_b):
    del items_factors

    def rowpad(x, w):
        return jnp.pad(x.astype(_F32), ((0, 0), (0, 128 - w)))

    pa = jnp.concatenate([
        items_feature_posi.reshape(32, 128),
        rowpad(outfit_items_posi, 8),
        rowpad(items_neighbor_posi.reshape(32, 4), 4),
        cf_W1.reshape(128, 128),
        jnp.concatenate([cf_W2[0], cf_W2[1]], axis=1),
        f2f_W1, f2f_W2, f2i_W1, f2i_W2, i2i_W1, i2i_W2,
        o2s_W,
        cf_b1.reshape(1, 128),
        cf_b2,
        f2f_b1.reshape(1, 128), f2f_b2.reshape(1, 128),
        f2i_b1.reshape(1, 128), f2i_b2.reshape(1, 128),
        i2i_b1.reshape(1, 128), i2i_b2.reshape(1, 128),
        rowpad(o2s_b.reshape(1, 1), 1),
        jnp.zeros((1, 128), _F32),
    ], axis=0)
    res = pl.pallas_call(
        _body,
        out_shape=jax.ShapeDtypeStruct((8, 128), _F32),
    )(pa)
    return res[:4, 0], res[4, 0:1]


# speedup vs baseline: 2.0434x; 2.0434x over previous
import jax
import jax.numpy as jnp
from jax.experimental import pallas as pl

_F32 = jnp.float32
_DEF = jax.lax.Precision.DEFAULT


def _leaky(v):
    return jnp.where(v >= 0, v, 0.01 * v)


def _mm(a, b):
    return jax.lax.dot_general(a, b, (((1,), (0,)), ((), ())),
                               precision=_DEF, preferred_element_type=_F32)


def _mmR(a, w):
    return jax.lax.dot_general(a, w, (((1,), (1,)), ((), ())),
                               precision=_DEF, preferred_element_type=_F32)


def _mmT(a, b):
    return jax.lax.dot_general(a, b, (((0,), (0,)), ((), ())),
                               precision=_DEF, preferred_element_type=_F32)


def _mlpR(x, w1, b1, w2, b2):
    h = _leaky(_mmR(x, w1) + b1)
    return _leaky(_mmR(h, w2) + b2)


def _body(oi_ref, nbr_ref, x0_ref, cfw1_ref, cfb1_ref, cfw2_ref, cfb2_ref,
          fw1_ref, fb1_ref, fw2_ref, fb2_ref,
          gw1_ref, gb1_ref, gw2_ref, gb2_ref,
          iw1_ref, ib1_ref, iw2_ref, ib2_ref,
          osw_ref, osb_ref, out_ref):
    X0 = x0_ref[:]
    oif = oi_ref[:].astype(_F32)
    nbrf = nbr_ref[:].astype(_F32)

    fw1 = fw1_ref[:]
    fw2 = fw2_ref[:]
    gw1 = gw1_ref[:]
    gw2 = gw2_ref[:]
    iw1 = iw1_ref[:]
    iw2 = iw2_ref[:]
    fb1 = fb1_ref[:]
    fb2 = fb2_ref[:]
    gb1 = gb1_ref[:]
    gb2 = gb2_ref[:]
    ib1 = ib1_ref[:]
    ib2 = ib2_ref[:]

    h = _leaky(_mmR(X0, cfw1_ref[:]) + cfb1_ref[:])
    F0 = _leaky(_mmR(h[:, 0:64], cfw2_ref[0]) + cfb2_ref[0:1, :])
    F1 = _leaky(_mmR(h[:, 64:128], cfw2_ref[1]) + cfb2_ref[1:2, :])
    F = jnp.concatenate([F0, F1], axis=1)

    _i32 = jnp.int32
    k_col = jax.lax.broadcasted_iota(_i32, (4, 1), 0).astype(_F32) * 8.0
    g_oi = oif + k_col
    iota483 = jax.lax.broadcasted_iota(_i32, (4, 8, 32), 2).astype(_F32)
    O32 = (iota483 == g_oi[:, :, None]).astype(_F32).reshape(32, 32)
    iota432 = jax.lax.broadcasted_iota(_i32, (4, 32), 1).astype(_F32)

    nb_all = _mm(O32, nbrf)
    Gs, moAs, moBs, Hps = [], [], [], []
    mAeq, mBeq, mTeq = [None], [None], [None]
    nb_pads = []
    for p in range(8):
        ip = oif[:, p:p + 1]
        gp = g_oi[:, p:p + 1]
        Hp = (iota432 == gp).astype(_F32)
        Sp = (iota432 == k_col + float(p)).astype(_F32)
        nb_sel = _mm(Sp, nb_all)
        nb_pad = jnp.concatenate(
            [nb_sel, jnp.full((4, 4), -100.0, _F32)], axis=1)
        g_nb = nb_pad + k_col
        Nb = (iota483 == g_nb[:, :, None]).astype(_F32).reshape(32, 32)
        Gs.append(jnp.concatenate([O32, Nb, Hp], axis=0))
        Hps.append(Hp)
        nb_pads.append(nb_pad)
        moAs.append((oif != ip).astype(_F32))
        moBs.append(((nb_pad != ip) & (nb_pad >= 0)).astype(_F32))
    for p in range(1, 8):
        ipm1 = oif[:, p - 1:p]
        mAeq.append((oif == ipm1).astype(_F32))
        mBeq.append((nb_pads[p] == ipm1).astype(_F32))
        mTeq.append((oif[:, p:p + 1] == ipm1).astype(_F32))

    Fstate = F
    Rpre = _mm(Gs[0], F)
    d_prev = None
    for p in range(8):
        if p == 0:
            A = Rpre[0:32]
            B = Rpre[32:64]
            T = Rpre[64:68]
        else:
            dbc = d_prev[:, None, :]
            A = Rpre[0:32] + (mAeq[p][:, :, None] * dbc).reshape(32, 256)
            B = Rpre[32:64] + (mBeq[p][:, :, None] * dbc).reshape(32, 256)
            T = Rpre[64:68] + mTeq[p] * d_prev
        Te = T[:, None, :]
        Xa = (Te * A.reshape(4, 8, 256)).reshape(32, 256)
        Xb = (Te * B.reshape(4, 8, 256)).reshape(32, 256)
        X = jnp.concatenate([Xa, Xb], axis=0)
        Xj = jnp.concatenate([X[:, :128], X[:, 128:]], axis=0)
        U = _mlpR(Xj, fw1, fb1, fw2, fb2)

        moA = moAs[p]
        moB = moBs[p]
        d0 = (jnp.sum(U[0:32].reshape(4, 8, 128) * moA[:, :, None], axis=1)
              + jnp.sum(U[32:64].reshape(4, 8, 128) * moB[:, :, None], axis=1))
        d1 = (jnp.sum(U[64:96].reshape(4, 8, 128) * moA[:, :, None], axis=1)
              + jnp.sum(U[96:128].reshape(4, 8, 128) * moB[:, :, None], axis=1))
        d = jnp.concatenate([d0, d1], axis=1)
        if p >= 1:
            Fstate = Fstate + _mmT(Hps[p - 1], d_prev)
        if p < 7:
            Rpre = _mm(Gs[p + 1], Fstate)
        d_prev = d
    F = Fstate + _mmT(Hps[7], d_prev)

    X3 = jnp.concatenate([F[:, :128], F[:, 128:]], axis=0)
    U3 = _mlpR(X3, gw1, gb1, gw2, gb2)
    Usum = U3[0:32] + U3[32:64]
    cnt = _mmT(O32, jnp.ones((32, 1), _F32))
    Ff = X0 + cnt * Usum

    Ffstate = Ff
    Wgpre = _mm(O32, Ff)
    s_prev = None
    for p in range(8):
        if p == 0:
            Wg = Wgpre
        else:
            Wg = Wgpre + (mAeq[p][:, :, None]
                          * s_prev[:, None, :]).reshape(32, 128)
        V = _mlpR(Wg, iw1, ib1, iw2, ib2)
        s = jnp.sum(V.reshape(4, 8, 128) * moAs[p][:, :, None], axis=1)
        if p >= 1:
            Ffstate = Ffstate + _mmT(Hps[p - 1], s_prev)
        if p < 7:
            Wgpre = _mm(O32, Ffstate)
        s_prev = s
    Ff = Ffstate + _mmT(Hps[7], s_prev)

    C = jnp.sum(O32.reshape(4, 8, 32), axis=1)
    outf = _mm(C, Ff)
    logits = (jnp.sum(outf * osw_ref[:], axis=1, keepdims=True)
              + osb_ref[0, 0])
    sc = jax.nn.sigmoid(logits)

    a = F[:, :128]
    b = F[:, 128:]
    aa = jnp.sum(a * a, axis=1, keepdims=True)
    ab = jnp.sum(a * b, axis=1, keepdims=True)
    bb = jnp.sum(b * b, axis=1, keepdims=True)
    terms = (aa - 1.0) ** 2 + 2.0 * ab * ab + (bb - 1.0) ** 2
    loss = jnp.sum(terms, axis=0, keepdims=True) / 4.0

    out_ref[:] = jnp.concatenate(
        [jnp.broadcast_to(sc, (4, 128)), jnp.broadcast_to(loss, (4, 128))],
        axis=0)


def kernel(outfit_items_posi, items_feature_posi, items_neighbor_posi,
           items_factors,
           cf_W1, cf_b1, cf_W2, cf_b2,
           f2f_W1, f2f_b1, f2f_W2, f2f_b2,
           f2i_W1, f2i_b1, f2i_W2, f2i_b2,
           i2i_W1, i2i_b1, i2i_W2, i2i_b2,
           o2s_W, o2s_b):
    del items_factors
    args = (
        outfit_items_posi,
        items_neighbor_posi.reshape(32, 4),
        items_feature_posi.reshape(32, 128),
        cf_W1.reshape(128, 128),
        cf_b1.reshape(1, 128),
        cf_W2,
        cf_b2,
        f2f_W1, f2f_b1.reshape(1, 128), f2f_W2, f2f_b2.reshape(1, 128),
        f2i_W1, f2i_b1.reshape(1, 128), f2i_W2, f2i_b2.reshape(1, 128),
        i2i_W1, i2i_b1.reshape(1, 128), i2i_W2, i2i_b2.reshape(1, 128),
        o2s_W, o2s_b.reshape(1, 1),
    )
    res = pl.pallas_call(
        _body,
        out_shape=jax.ShapeDtypeStruct((8, 128), _F32),
    )(*args)
    return res[:4, 0], res[4, 0:1]
